# baseline (speedup 1.0000x reference)
import functools

import jax
import jax.numpy as jnp
from jax import lax
from jax.experimental import pallas as pl
from jax.experimental.pallas import tpu as pltpu
from jax.experimental.pallas import tpu_sc as plsc


def _sc_gather_tt(action_1d, current_1d, ttm_t, B, N):
    info = plsc.get_sparse_core_info()
    NC, NS = info.num_cores, info.num_subcores
    NW = NC * NS
    assert B % NW == 0
    bpw = B // NW
    assert bpw % 16 == 0

    mesh = plsc.VectorSubcoreMesh(core_axis_name="c", subcore_axis_name="s")

    @functools.partial(
        pl.kernel,
        out_type=jax.ShapeDtypeStruct((B,), jnp.float32),
        mesh=mesh,
        scratch_types=[
            pltpu.VMEM((bpw,), jnp.int32),
            pltpu.VMEM((bpw,), jnp.int32),
            pltpu.VMEM((bpw, 16), jnp.float32),
            pltpu.VMEM((bpw,), jnp.float32),
            pltpu.SemaphoreType.DMA,
            pltpu.SemaphoreType.DMA,
        ],
    )
    def sc_kernel(act_hbm, cur_hbm, ttm_hbm, tt_hbm,
                  act_v, cur_v, win_v, tt_v, sem_in, sem_g):
        wid = lax.axis_index("s") * NC + lax.axis_index("c")
        base = wid * bpw
        sl_all = pl.ds(base, bpw)
        cp_a = pltpu.async_copy(act_hbm.at[sl_all], act_v, sem_in)
        cp_c = pltpu.async_copy(cur_hbm.at[sl_all], cur_v, sem_in)
        cp_a.wait()
        cp_c.wait()

        cps = []
        for j in range(bpw // 16):
            av = act_v[pl.ds(j * 16, 16)]
            cv = cur_v[pl.ds(j * 16, 16)]
            lanes = pl.ds(base + j * 16, 16)
            for k in range(16):
                i = j * 16 + k
                a = av[k]
                c = cv[k]
                cps.append(pltpu.async_copy(ttm_hbm.at[c, a, lanes],
                                            win_v.at[i], sem_g))
        for cp in cps:
            cp.wait()

        lane = lax.iota(jnp.int32, 16)
        for j in range(bpw // 16):
            acc = jnp.zeros((16,), jnp.float32)
            for k in range(16):
                acc = jnp.where(lane == k, win_v[pl.ds(j * 16 + k, 1), :][0],
                                acc)
            tt_v[pl.ds(j * 16, 16)] = acc

        pltpu.sync_copy(tt_v, tt_hbm.at[sl_all])

    return sc_kernel(action_1d, current_1d, ttm_t)


def _tc_select_body(act_ref, tw0_ref, dm_ref, comp_ref,
                    sw_ref, dmsel_ref, out_ref):
    a = act_ref[...]
    comp = comp_ref[...]
    row = lax.broadcasted_iota(jnp.int32, comp.shape, 0)
    onehot = row == a
    sw_ref[...] = jnp.sum(jnp.where(onehot, tw0_ref[...], 0.0), axis=0,
                          keepdims=True)
    dmsel_ref[...] = jnp.sum(jnp.where(onehot, dm_ref[...], 0.0), axis=0,
                             keepdims=True)
    is_drop = (a % 2 == 0) & (a != 0)
    hit = onehot | (row == a - 1)
    out_ref[...] = comp | (is_drop & hit).astype(jnp.int8)


def _tc_math_body(act_ref, cur_ref, ct_ref, uc_ref, tt_ref, sw_ref, dm_ref,
                  sst_ref, nl_ref):
    a = act_ref[...]
    cur = cur_ref[...]
    sst = jnp.maximum(ct_ref[...] + tt_ref[...], sw_ref[...])
    is_ret = (a == 0) & (cur != 0)
    sst_ref[...] = jnp.where(is_ret, 0.0, sst)
    nl_ref[...] = jnp.where(is_ret, 0.0, uc_ref[...] + dm_ref[...])


def kernel(action, current_node, current_time, used_capacity,
           travel_time_matrix, time_windows, demand, completed):
    B = action.shape[0]
    N = travel_time_matrix.shape[1]

    act1 = action.astype(jnp.int32)
    cur1 = current_node.reshape(B).astype(jnp.int32)

    ttm_t = jnp.transpose(travel_time_matrix, (1, 2, 0))
    tw0_t = jnp.transpose(time_windows, (2, 1, 0))[0]
    dm_t = jnp.transpose(demand, (1, 0))
    comp_t = jnp.transpose(completed.view(jnp.int8), (1, 0))

    tt = _sc_gather_tt(act1, cur1, ttm_t, B, N)

    bcols = 2048
    grid = B // bcols
    row_spec = pl.BlockSpec((1, bcols), lambda i: (0, i))
    mat_spec = pl.BlockSpec((N, bcols), lambda i: (0, i))
    act_row = act1.reshape(1, B)
    cur_row = cur1.reshape(1, B)
    sw_r, dm_r, comp_out_t = pl.pallas_call(
        _tc_select_body,
        grid=(grid,),
        in_specs=[
            row_spec,
            mat_spec,
            mat_spec,
            mat_spec,
        ],
        out_specs=[row_spec, row_spec, mat_spec],
        out_shape=[
            jax.ShapeDtypeStruct((1, B), jnp.float32),
            jax.ShapeDtypeStruct((1, B), jnp.float32),
            jax.ShapeDtypeStruct((N, B), jnp.int8),
        ],
    )(act_row, tw0_t, dm_t, comp_t)

    full_row = pl.BlockSpec((1, B), lambda: (0, 0))
    sst_r, nl_r = pl.pallas_call(
        _tc_math_body,
        in_specs=[full_row] * 7,
        out_specs=[full_row, full_row],
        out_shape=[
            jax.ShapeDtypeStruct((1, B), jnp.float32),
            jax.ShapeDtypeStruct((1, B), jnp.float32),
        ],
    )(act_row, cur_row, current_time.reshape(1, B),
      used_capacity.reshape(1, B), tt.reshape(1, B), sw_r, dm_r)

    return (sst_r.reshape(B, 1), nl_r.reshape(B, 1),
            jnp.transpose(comp_out_t, (1, 0)).view(jnp.bool_))


# speedup vs baseline: 1.0466x; 1.0466x over previous
import functools

import jax
import jax.numpy as jnp
from jax import lax
from jax.experimental import pallas as pl
from jax.experimental.pallas import tpu as pltpu
from jax.experimental.pallas import tpu_sc as plsc


def _sc_gather_tt(action_1d, current_1d, ttm_t, B, N):
    info = plsc.get_sparse_core_info()
    NC, NS = info.num_cores, info.num_subcores
    NW = NC * NS
    assert B % NW == 0
    bpw = B // NW
    assert bpw % 16 == 0

    mesh = plsc.VectorSubcoreMesh(core_axis_name="c", subcore_axis_name="s")

    @functools.partial(
        pl.kernel,
        out_type=jax.ShapeDtypeStruct((B,), jnp.float32),
        mesh=mesh,
        scratch_types=[
            pltpu.VMEM((bpw,), jnp.int32),
            pltpu.VMEM((bpw,), jnp.int32),
            pltpu.VMEM((bpw * 16,), jnp.float32),
            pltpu.VMEM((bpw,), jnp.float32),
            pltpu.SemaphoreType.DMA,
            pltpu.SemaphoreType.DMA,
        ],
    )
    def sc_kernel(act_hbm, cur_hbm, ttm_hbm, tt_hbm,
                  act_v, cur_v, win_v, tt_v, sem_in, sem_g):
        wid = lax.axis_index("s") * NC + lax.axis_index("c")
        base = wid * bpw
        sl_all = pl.ds(base, bpw)
        cp_a = pltpu.async_copy(act_hbm.at[sl_all], act_v, sem_in)
        cp_c = pltpu.async_copy(cur_hbm.at[sl_all], cur_v, sem_in)
        cp_a.wait()
        cp_c.wait()

        for j in range(bpw // 16):
            av = act_v[pl.ds(j * 16, 16)]
            cv = cur_v[pl.ds(j * 16, 16)]
            lanes = pl.ds(base + j * 16, 16)
            for k in range(16):
                i = j * 16 + k
                a = av[k]
                c = cv[k]
                pltpu.async_copy(ttm_hbm.at[c, a, lanes],
                                 win_v.at[pl.ds(i * 16, 16)], sem_g)
        pltpu.make_async_copy(tt_hbm.at[pl.ds(0, bpw * 16)], win_v,
                              sem_g).wait()

        lane = lax.iota(jnp.int32, 16)
        for j in range(bpw // 16):
            acc = jnp.zeros((16,), jnp.float32)
            for k in range(16):
                i = j * 16 + k
                acc = jnp.where(lane == k, win_v[pl.ds(i * 16, 16)], acc)
            tt_v[pl.ds(j * 16, 16)] = acc

        pltpu.sync_copy(tt_v, tt_hbm.at[sl_all])

    return sc_kernel(action_1d, current_1d, ttm_t)


def _tc_select_body(act_ref, tw0_ref, dm_ref, comp_ref,
                    sw_ref, dmsel_ref, out_ref):
    a = act_ref[...]
    comp = comp_ref[...]
    row = lax.broadcasted_iota(jnp.int32, comp.shape, 0)
    onehot = row == a
    sw_ref[...] = jnp.sum(jnp.where(onehot, tw0_ref[...], 0.0), axis=0,
                          keepdims=True)
    dmsel_ref[...] = jnp.sum(jnp.where(onehot, dm_ref[...], 0.0), axis=0,
                             keepdims=True)
    is_drop = (a % 2 == 0) & (a != 0)
    hit = onehot | (row == a - 1)
    out_ref[...] = comp | (is_drop & hit).astype(jnp.int8)


def _tc_math_body(act_ref, cur_ref, ct_ref, uc_ref, tt_ref, sw_ref, dm_ref,
                  sst_ref, nl_ref):
    a = act_ref[...]
    cur = cur_ref[...]
    sst = jnp.maximum(ct_ref[...] + tt_ref[...], sw_ref[...])
    is_ret = (a == 0) & (cur != 0)
    sst_ref[...] = jnp.where(is_ret, 0.0, sst)
    nl_ref[...] = jnp.where(is_ret, 0.0, uc_ref[...] + dm_ref[...])


def kernel(action, current_node, current_time, used_capacity,
           travel_time_matrix, time_windows, demand, completed):
    B = action.shape[0]
    N = travel_time_matrix.shape[1]

    act1 = action.astype(jnp.int32)
    cur1 = current_node.reshape(B).astype(jnp.int32)

    ttm_t = jnp.transpose(travel_time_matrix, (1, 2, 0))
    tw0_t = jnp.transpose(time_windows[:, :, 0], (1, 0))
    dm_t = jnp.transpose(demand, (1, 0))
    comp_t = jnp.transpose(completed.view(jnp.int8), (1, 0))

    tt = _sc_gather_tt(act1, cur1, ttm_t, B, N)

    bcols = 2048
    grid = B // bcols
    row_spec = pl.BlockSpec((1, bcols), lambda i: (0, i))
    mat_spec = pl.BlockSpec((N, bcols), lambda i: (0, i))
    act_row = act1.reshape(1, B)
    cur_row = cur1.reshape(1, B)
    sw_r, dm_r, comp_out_t = pl.pallas_call(
        _tc_select_body,
        grid=(grid,),
        in_specs=[
            row_spec,
            mat_spec,
            mat_spec,
            mat_spec,
        ],
        out_specs=[row_spec, row_spec, mat_spec],
        out_shape=[
            jax.ShapeDtypeStruct((1, B), jnp.float32),
            jax.ShapeDtypeStruct((1, B), jnp.float32),
            jax.ShapeDtypeStruct((N, B), jnp.int8),
        ],
    )(act_row, tw0_t, dm_t, comp_t)

    full_row = pl.BlockSpec((1, B), lambda: (0, 0))
    sst_r, nl_r = pl.pallas_call(
        _tc_math_body,
        in_specs=[full_row] * 7,
        out_specs=[full_row, full_row],
        out_shape=[
            jax.ShapeDtypeStruct((1, B), jnp.float32),
            jax.ShapeDtypeStruct((1, B), jnp.float32),
        ],
    )(act_row, cur_row, current_time.reshape(1, B),
      used_capacity.reshape(1, B), tt.reshape(1, B), sw_r, dm_r)

    return (sst_r.reshape(B, 1), nl_r.reshape(B, 1),
            jnp.transpose(comp_out_t, (1, 0)).view(jnp.bool_))
